# Initial kernel scaffold; baseline (speedup 1.0000x reference)
import jax
import jax.numpy as jnp
from jax.experimental import pallas as pl


def kernel(x_transaction, x_user, Win_tx, bin_tx, Win_us, bin_us, W1_ut, as1_ut, ad1_ut, b1_ut, W1_tu, as1_tu, ad1_tu, b1_tu, W2_ut, as2_ut, ad2_ut, b2_ut, W2_tu, as2_tu, ad2_tu, b2_tu, Wc, bc, ei_user_to_tx, ei_tx_to_user):
    raise NotImplementedError("write your pallas kernel here")


# speedup vs baseline: 8.2166x; 8.2166x over previous
import jax
import jax.numpy as jnp
from jax import lax
from jax.experimental import pallas as pl
from jax.experimental.pallas import tpu as pltpu
from jax.experimental.pallas import tpu_sc as plsc

N = 10000
E = 320000
D_IN, HID, HEADS, OUT = 128, 64, 4, 32

NC, NS, L = 2, 16, 16
CH = 80
RB = 2000

_f32 = jnp.float32
_i32 = jnp.int32

_sc_params = pltpu.CompilerParams(needs_layout_passes=False,
                                  use_tc_tiling_on_sc=False)
_sc_mesh = plsc.VectorSubcoreMesh(core_axis_name="c", subcore_axis_name="s",
                                  num_cores=NC, num_subcores=NS)


def _lrelu(x):
    return jnp.maximum(x, 0.2 * x)


def _elu(x):
    return jnp.where(x > 0, x, jnp.exp(jnp.minimum(x, 0.0)) - 1.0)


def _stage_a1_body(x_tx, x_us, win_tx, bin_tx, win_us, bin_us,
                   w1_ut, as1_ut, ad1_ut, w1_tu, as1_tu, ad1_tu,
                   o_hs_ut, o_as_ut, o_ad_ut, o_hs_tu, o_as_tu, o_ad_tu):
    h_tx = _elu(jnp.dot(x_tx[...], win_tx[...],
                        preferred_element_type=_f32) + bin_tx[...])
    h_us = _elu(jnp.dot(x_us[...], win_us[...],
                        preferred_element_type=_f32) + bin_us[...])

    def direction(h_src, h_dst, w, att_s, att_d, o_hs, o_as, o_ad):
        hs = jnp.dot(h_src, w[...], preferred_element_type=_f32)
        hd = jnp.dot(h_dst, w[...], preferred_element_type=_f32)
        a_s_cols, a_d_cols = [], []
        for h in range(HEADS):
            sl = slice(h * HID, (h + 1) * HID)
            o_hs[h, :, :] = hs[:, sl]
            a_s_cols.append(jnp.sum(hs[:, sl] * att_s[h:h + 1, :],
                                    axis=1, keepdims=True))
            a_d_cols.append(jnp.sum(hd[:, sl] * att_d[h:h + 1, :],
                                    axis=1, keepdims=True))
        o_as[...] = jnp.concatenate(a_s_cols, axis=1)
        o_ad[...] = jnp.concatenate(a_d_cols, axis=1)

    direction(h_us, h_tx, w1_ut, as1_ut[...], ad1_ut[...],
              o_hs_ut, o_as_ut, o_ad_ut)
    direction(h_tx, h_us, w1_tu, as1_tu[...], ad1_tu[...],
              o_hs_tu, o_as_tu, o_ad_tu)


_stage_a1 = pl.pallas_call(
    _stage_a1_body,
    grid=(N // RB,),
    in_specs=[
        pl.BlockSpec((RB, D_IN), lambda b: (b, 0)),
        pl.BlockSpec((RB, D_IN), lambda b: (b, 0)),
        pl.BlockSpec((D_IN, HID), lambda b: (0, 0)),
        pl.BlockSpec((1, HID), lambda b: (0, 0)),
        pl.BlockSpec((D_IN, HID), lambda b: (0, 0)),
        pl.BlockSpec((1, HID), lambda b: (0, 0)),
        pl.BlockSpec((HID, HEADS * HID), lambda b: (0, 0)),
        pl.BlockSpec((HEADS, HID), lambda b: (0, 0)),
        pl.BlockSpec((HEADS, HID), lambda b: (0, 0)),
        pl.BlockSpec((HID, HEADS * HID), lambda b: (0, 0)),
        pl.BlockSpec((HEADS, HID), lambda b: (0, 0)),
        pl.BlockSpec((HEADS, HID), lambda b: (0, 0)),
    ],
    out_specs=(
        pl.BlockSpec((HEADS, RB, HID), lambda b: (0, b, 0)),
        pl.BlockSpec((RB, HEADS), lambda b: (b, 0)),
        pl.BlockSpec((RB, HEADS), lambda b: (b, 0)),
        pl.BlockSpec((HEADS, RB, HID), lambda b: (0, b, 0)),
        pl.BlockSpec((RB, HEADS), lambda b: (b, 0)),
        pl.BlockSpec((RB, HEADS), lambda b: (b, 0)),
    ),
    out_shape=(
        jax.ShapeDtypeStruct((HEADS, N, HID), _f32),
        jax.ShapeDtypeStruct((N, HEADS), _f32),
        jax.ShapeDtypeStruct((N, HEADS), _f32),
        jax.ShapeDtypeStruct((HEADS, N, HID), _f32),
        jax.ShapeDtypeStruct((N, HEADS), _f32),
        jax.ShapeDtypeStruct((N, HEADS), _f32),
    ),
)


def _gmax_body(as_ut, as_tu, o_g):
    o_g[0:1, :] = jnp.max(as_ut[...], axis=0, keepdims=True)
    o_g[1:2, :] = jnp.max(as_tu[...], axis=0, keepdims=True)


_gmax = pl.pallas_call(
    _gmax_body,
    out_shape=jax.ShapeDtypeStruct((2, HEADS), _f32),
)


def _stage_a2(a_s, a_d, g, d):
    return pl.pallas_call(
        lambda a_s_ref, a_d_ref, g_ref, o_ref: _stage_a2_meta(
            a_s_ref, a_d_ref, g_ref, o_ref, d),
        grid=(HEADS, N // RB),
        in_specs=[
            pl.BlockSpec((RB, HEADS), lambda h, b: (b, 0)),
            pl.BlockSpec((RB, HEADS), lambda h, b: (b, 0)),
            pl.BlockSpec((2, HEADS), lambda h, b: (0, 0)),
        ],
        out_specs=pl.BlockSpec((RB, 16),
                               lambda h, b: (h * (N // RB) + b, 0)),
        out_shape=jax.ShapeDtypeStruct((HEADS * N, 16), _f32),
    )(a_s, a_d, g)


def _stage_a2_meta(a_s, a_d, g, o_meta, d):
    h = pl.program_id(0)
    asl = jnp.zeros((RB, 1), _f32)
    adl = jnp.zeros((RB, 1), _f32)
    gv = jnp.float32(0)
    for hh in range(HEADS):
        asl = jnp.where(h == hh, a_s[:, hh:hh + 1], asl)
        adl = jnp.where(h == hh, a_d[:, hh:hh + 1], adl)
        gv = jnp.where(h == hh, g[d, hh], gv)
    m = _lrelu(gv + adl)
    o_meta[...] = jnp.concatenate(
        [asl, adl, m, jnp.zeros((RB, 13), _f32)], axis=1)


def _gat1_body(hs_ut, meta_ut, src_ut, dst_ut, hs_tu, meta_tu, src_tu, dst_tu,
               out_ut, den_out_ut, out_tu, den_out_tu,
               idx_srcp, idx_dst, idx_dstp, idx_dstl, rows, meta_s, meta_d,
               wbuf, ebuf, zbuf, den, acc_sh, sem):
    cid = lax.axis_index("c")
    sid = lax.axis_index("s")
    epw = E // NS
    lane = lax.iota(_i32, L)
    zeros_i = jnp.zeros((L,), _i32)
    ones_i = jnp.full((L,), 1, _i32)
    zeros_f = jnp.zeros((L,), _f32)
    lane0 = lane == 0
    HN = N // 2
    lo = cid * HN

    def run_pass(hs_hbm, meta_hbm, src_hbm, dst_hbm, out_hbm, p):
        page_off = p * N

        def zrow(r, _):
            for j in range(4):
                zbuf[r, pl.ds(j * L, L)] = jnp.zeros((L,), _f32)
            return 0
        lax.fori_loop(0, 80, zrow, 0)
        for k in range(4):
            pltpu.sync_copy(zbuf, acc_sh.at[pl.ds(sid * 320 + k * 80, 80)])
        plsc.subcore_barrier()

        def chunk(ci, _):
            base = sid * epw + ci * CH
            pltpu.sync_copy(src_hbm.at[pl.ds(base, CH)], idx_srcp)
            pltpu.sync_copy(dst_hbm.at[pl.ds(base, CH)], idx_dst)

            def adj(g, _):
                s = idx_srcp[pl.ds(g * L, L)]
                d = idx_dst[pl.ds(g * L, L)]
                idx_srcp[pl.ds(g * L, L)] = s + page_off
                idx_dstp[pl.ds(g * L, L)] = d + page_off
                dl = d - lo
                ok = (dl >= 0) & (dl < HN)
                idx_dstl[pl.ds(g * L, L)] = jnp.where(ok, dl, HN)
                return 0
            lax.fori_loop(0, CH // L, adj, 0)

            d1 = pltpu.async_copy(hs_hbm.at[idx_srcp], rows, sem)
            d2 = pltpu.async_copy(meta_hbm.at[idx_srcp], meta_s, sem)
            d3 = pltpu.async_copy(meta_hbm.at[idx_dstp], meta_d, sem)
            d1.wait()
            d2.wait()
            d3.wait()

            def egrp(g, _):
                r16 = g * L + lane
                a_s = plsc.load_gather(meta_s, [r16, zeros_i])
                a_d = plsc.load_gather(meta_d, [r16, ones_i])
                m = plsc.load_gather(meta_d, [r16, jnp.full((L,), 2, _i32)])
                ebuf[0, pl.ds(g * L, L)] = jnp.exp(_lrelu(a_s + a_d) - m)
                return 0
            lax.fori_loop(0, CH // L, egrp, 0)

            pf = jnp.full((L,), p, _i32)

            def wedge(jj, _):
                for u in range(4):
                    j = jj * 4 + u
                    jf = jnp.full((L,), j, _i32)
                    dj = plsc.load_gather(idx_dst, [jf])
                    ok = (dj >= lo) & (dj < lo + HN)
                    e = jnp.where(ok, plsc.load_gather(ebuf, [zeros_i, jf]),
                                  0.0)
                    for k in range(4):
                        wbuf[j, pl.ds(k * L, L)] = rows[j, pl.ds(k * L, L)] * e
                    cur = plsc.load_gather(den, [dj, pf])
                    plsc.store_scatter(den, [dj, pf], cur + e,
                                       mask=lane0 & ok)
                return 0
            lax.fori_loop(0, CH // 4, wedge, 0)

            pltpu.sync_copy(wbuf, acc_sh.at[idx_dstl], add=True)
            return 0

        lax.fori_loop(0, epw // CH, chunk, 0)
        plsc.subcore_barrier()

        @pl.when(sid == 0)
        def _():
            pltpu.sync_copy(acc_sh.at[pl.ds(0, HN)],
                            out_hbm.at[pl.ds(p * N + cid * HN, HN)])
        plsc.subcore_barrier()

    def run_direction(hs_hbm, meta_hbm, src_hbm, dst_hbm, out_hbm, den_hbm):
        def zden(i, _):
            f = i * L + lane
            plsc.store_scatter(den, [jax.lax.shift_right_logical(f, 2),
                                     jnp.bitwise_and(f, 3)], zeros_f)
            return 0
        lax.fori_loop(0, (N * 4) // L, zden, 0)

        for p in range(HEADS):
            run_pass(hs_hbm, meta_hbm, src_hbm, dst_hbm, out_hbm, p)

        pltpu.sync_copy(den, den_hbm.at[cid * NS + sid])

    run_direction(hs_ut, meta_ut, src_ut, dst_ut, out_ut, den_out_ut)
    run_direction(hs_tu, meta_tu, src_tu, dst_tu, out_tu, den_out_tu)


_gat1 = pl.kernel(
    _gat1_body,
    out_type=(
        jax.ShapeDtypeStruct((HEADS * N, HID), _f32),
        jax.ShapeDtypeStruct((NC * NS, N, HEADS), _f32),
        jax.ShapeDtypeStruct((HEADS * N, HID), _f32),
        jax.ShapeDtypeStruct((NC * NS, N, HEADS), _f32),
    ),
    mesh=_sc_mesh,
    compiler_params=_sc_params,
    scratch_types=[
        pltpu.VMEM((CH,), _i32),
        pltpu.VMEM((CH,), _i32),
        pltpu.VMEM((CH,), _i32),
        pltpu.VMEM((CH,), _i32),
        pltpu.VMEM((CH, HID), _f32),
        pltpu.VMEM((CH, 16), _f32),
        pltpu.VMEM((CH, 16), _f32),
        pltpu.VMEM((CH, HID), _f32),
        pltpu.VMEM((2, CH), _f32),
        pltpu.VMEM((80, HID), _f32),
        pltpu.VMEM((N, HEADS), _f32),
        pltpu.VMEM_SHARED((5120, HID), _f32),
        pltpu.SemaphoreType.DMA,
    ],
)


def _stage_b1_body(acc, den, b1, o_h):
    c_i = lax.broadcasted_iota(_i32, (HEADS * NC * NS, HEADS), 0)
    h_i = lax.broadcasted_iota(_i32, (HEADS * NC * NS, HEADS), 1)
    sel = jnp.where(h_i == c_i % HEADS, 1.0, 0.0)
    s = jnp.dot(den[...], sel, preferred_element_type=_f32)
    cols = []
    for h in range(HEADS):
        r = 1.0 / (s[:, h:h + 1] + 1e-16)
        cols.append(acc[h, :, :] * r)
    o_h[...] = _elu(jnp.concatenate(cols, axis=1) + b1[...])


_stage_b1 = pl.pallas_call(
    _stage_b1_body,
    grid=(N // RB,),
    in_specs=[
        pl.BlockSpec((HEADS, RB, HID), lambda b: (0, b, 0)),
        pl.BlockSpec((RB, HEADS * NC * NS), lambda b: (b, 0)),
        pl.BlockSpec((1, HEADS * HID), lambda b: (0, 0)),
    ],
    out_specs=pl.BlockSpec((RB, HEADS * HID), lambda b: (b, 0)),
    out_shape=jax.ShapeDtypeStruct((N, HEADS * HID), _f32),
)


def _stage_b2a_body(h_tx2, h_us2, w2, as2, ad2, o_hs2, o_as, o_ad):
    hs2 = jnp.dot(h_us2[...], w2[...], preferred_element_type=_f32)
    hd2 = jnp.dot(h_tx2[...], w2[...], preferred_element_type=_f32)
    o_hs2[...] = hs2
    o_as[...] = jnp.sum(hs2 * as2[...], axis=1, keepdims=True)
    o_ad[...] = jnp.sum(hd2 * ad2[...], axis=1, keepdims=True)


_stage_b2a = pl.pallas_call(
    _stage_b2a_body,
    grid=(N // RB,),
    in_specs=[
        pl.BlockSpec((RB, HEADS * HID), lambda b: (b, 0)),
        pl.BlockSpec((RB, HEADS * HID), lambda b: (b, 0)),
        pl.BlockSpec((HEADS * HID, OUT), lambda b: (0, 0)),
        pl.BlockSpec((1, OUT), lambda b: (0, 0)),
        pl.BlockSpec((1, OUT), lambda b: (0, 0)),
    ],
    out_specs=(
        pl.BlockSpec((RB, OUT), lambda b: (b, 0)),
        pl.BlockSpec((RB, 1), lambda b: (b, 0)),
        pl.BlockSpec((RB, 1), lambda b: (b, 0)),
    ),
    out_shape=(
        jax.ShapeDtypeStruct((N, OUT), _f32),
        jax.ShapeDtypeStruct((N, 1), _f32),
        jax.ShapeDtypeStruct((N, 1), _f32),
    ),
)


def _gmax2_body(a_s, o_g):
    o_g[...] = jnp.max(a_s[...], axis=0, keepdims=True)


_gmax2 = pl.pallas_call(
    _gmax2_body,
    out_shape=jax.ShapeDtypeStruct((1, 1), _f32),
)


def _stage_b2c_body(hs2, a_s, a_d, g, o_hs2, o_meta2):
    gv = g[0, 0]
    m = _lrelu(gv + a_d[...])
    o_hs2[...] = jnp.concatenate(
        [hs2[...], a_s[...], jnp.zeros((RB, HID - OUT - 1), _f32)], axis=1)
    o_meta2[...] = jnp.concatenate(
        [a_d[...], m, jnp.zeros((RB, 14), _f32)], axis=1)


_stage_b2c = pl.pallas_call(
    _stage_b2c_body,
    grid=(N // RB,),
    in_specs=[
        pl.BlockSpec((RB, OUT), lambda b: (b, 0)),
        pl.BlockSpec((RB, 1), lambda b: (b, 0)),
        pl.BlockSpec((RB, 1), lambda b: (b, 0)),
        pl.BlockSpec((1, 1), lambda b: (0, 0)),
    ],
    out_specs=(
        pl.BlockSpec((RB, HID), lambda b: (b, 0)),
        pl.BlockSpec((RB, 16), lambda b: (b, 0)),
    ),
    out_shape=(
        jax.ShapeDtypeStruct((N, HID), _f32),
        jax.ShapeDtypeStruct((N, 16), _f32),
    ),
)


def _gat2_body(hs_hbm, meta_hbm, src_hbm, dst_hbm, out_hbm, den_hbm,
               idx_src, idx_dst, idx_dstl, rows, meta_d, wbuf, ebuf, zbuf,
               den, acc_sh, sem):
    cid = lax.axis_index("c")
    sid = lax.axis_index("s")
    epw = E // NS
    lane = lax.iota(_i32, L)
    zeros_i = jnp.zeros((L,), _i32)
    ones_i = jnp.full((L,), 1, _i32)
    zeros_f = jnp.zeros((L,), _f32)
    lane0 = lane == 0
    lo = cid * (N // 2)

    def zrow(r, _):
        for j in range(2):
            zbuf[r, pl.ds(j * L, L)] = jnp.zeros((L,), _f32)
        return 0
    lax.fori_loop(0, 80, zrow, 0)
    for k in range(4):
        pltpu.sync_copy(zbuf, acc_sh.at[pl.ds(sid * 320 + k * 80, 80)])

    def zden(i, _):
        f = i * L + lane
        plsc.store_scatter(den, [jax.lax.shift_right_logical(f, 1),
                                 jnp.bitwise_and(f, 1)], zeros_f)
        return 0
    lax.fori_loop(0, (N * 2) // L, zden, 0)
    plsc.subcore_barrier()

    def chunk(ci, _):
        base = sid * epw + ci * CH
        pltpu.sync_copy(src_hbm.at[pl.ds(base, CH)], idx_src)
        pltpu.sync_copy(dst_hbm.at[pl.ds(base, CH)], idx_dst)

        def adj(g, _):
            d = idx_dst[pl.ds(g * L, L)]
            dl = d - lo
            ok = (dl >= 0) & (dl < N // 2)
            idx_dstl[pl.ds(g * L, L)] = jnp.where(ok, dl, N // 2)
            return 0
        lax.fori_loop(0, CH // L, adj, 0)

        d1 = pltpu.async_copy(hs_hbm.at[idx_src], rows, sem)
        d2 = pltpu.async_copy(meta_hbm.at[idx_dst], meta_d, sem)
        d1.wait()
        d2.wait()

        def egrp(g, _):
            r16 = g * L + lane
            a_s = plsc.load_gather(rows, [r16, jnp.full((L,), OUT, _i32)])
            a_d = plsc.load_gather(meta_d, [r16, zeros_i])
            m = plsc.load_gather(meta_d, [r16, ones_i])
            ebuf[0, pl.ds(g * L, L)] = jnp.exp(_lrelu(a_s + a_d) - m)
            return 0
        lax.fori_loop(0, CH // L, egrp, 0)

        def wedge(jj, _):
            for u in range(4):
                j = jj * 4 + u
                jf = jnp.full((L,), j, _i32)
                dj = plsc.load_gather(idx_dst, [jf])
                ok = (dj >= lo) & (dj < lo + N // 2)
                e = jnp.where(ok, plsc.load_gather(ebuf, [zeros_i, jf]), 0.0)
                for k in range(2):
                    wbuf[j, pl.ds(k * L, L)] = rows[j, pl.ds(k * L, L)] * e
                cur = plsc.load_gather(den, [dj, zeros_i])
                plsc.store_scatter(den, [dj, zeros_i], cur + e,
                                   mask=lane0 & ok)
            return 0
        lax.fori_loop(0, CH // 4, wedge, 0)

        pltpu.sync_copy(wbuf, acc_sh.at[idx_dstl], add=True)
        return 0

    lax.fori_loop(0, epw // CH, chunk, 0)
    plsc.subcore_barrier()

    pltpu.sync_copy(den, den_hbm.at[cid * NS + sid])

    @pl.when(sid == 0)
    def _():
        pltpu.sync_copy(acc_sh.at[pl.ds(0, N // 2)],
                        out_hbm.at[pl.ds(cid * (N // 2), N // 2)])


_gat2 = pl.kernel(
    _gat2_body,
    out_type=(
        jax.ShapeDtypeStruct((N, OUT), _f32),
        jax.ShapeDtypeStruct((NC * NS, N, 2), _f32),
    ),
    mesh=_sc_mesh,
    compiler_params=_sc_params,
    scratch_types=[
        pltpu.VMEM((CH,), _i32),
        pltpu.VMEM((CH,), _i32),
        pltpu.VMEM((CH,), _i32),
        pltpu.VMEM((CH, HID), _f32),
        pltpu.VMEM((CH, 16), _f32),
        pltpu.VMEM((CH, OUT), _f32),
        pltpu.VMEM((2, CH), _f32),
        pltpu.VMEM((80, OUT), _f32),
        pltpu.VMEM((N, 2), _f32),
        pltpu.VMEM_SHARED((5120, OUT), _f32),
        pltpu.SemaphoreType.DMA,
    ],
)


def _stage_c_body(acc2, den2, b2, wc, bc, o):
    c_i = lax.broadcasted_iota(_i32, (2 * NC * NS, 1), 0)
    sel = jnp.where(c_i % 2 == 0, 1.0, 0.0)
    s = jnp.dot(den2[...], sel, preferred_element_type=_f32)
    a = acc2[...]
    t2 = a / (s + 1e-16) + b2[...]
    h = _elu(t2)
    o[...] = jnp.sum(h * wc[...], axis=1, keepdims=True) + bc[...]


_stage_c = pl.pallas_call(
    _stage_c_body,
    out_shape=jax.ShapeDtypeStruct((N, 1), _f32),
)


def kernel(x_transaction, x_user, Win_tx, bin_tx, Win_us, bin_us,
           W1_ut, as1_ut, ad1_ut, b1_ut, W1_tu, as1_tu, ad1_tu, b1_tu,
           W2_ut, as2_ut, ad2_ut, b2_ut, W2_tu, as2_tu, ad2_tu, b2_tu,
           Wc, bc, ei_user_to_tx, ei_tx_to_user):
    src_ut = ei_user_to_tx[0].astype(_i32)
    dst_ut = ei_user_to_tx[1].astype(_i32)
    src_tu = ei_tx_to_user[0].astype(_i32)
    dst_tu = ei_tx_to_user[1].astype(_i32)

    hs_ut, as_ut, ad_ut, hs_tu, as_tu, ad_tu = _stage_a1(
        x_transaction, x_user, Win_tx, bin_tx.reshape(1, HID),
        Win_us, bin_us.reshape(1, HID),
        W1_ut, as1_ut, ad1_ut, W1_tu, as1_tu, ad1_tu)
    g1 = _gmax(as_ut, as_tu)
    meta_ut = _stage_a2(as_ut, ad_ut, g1, 0)
    meta_tu = _stage_a2(as_tu, ad_tu, g1, 1)

    acc_ut, den_ut, acc_tu, den_tu = _gat1(
        hs_ut.reshape(HEADS * N, HID), meta_ut, src_ut, dst_ut,
        hs_tu.reshape(HEADS * N, HID), meta_tu, src_tu, dst_tu)
    den_ut = den_ut.transpose(1, 0, 2).reshape(N, HEADS * NC * NS)
    den_tu = den_tu.transpose(1, 0, 2).reshape(N, HEADS * NC * NS)

    h_tx2 = _stage_b1(acc_ut.reshape(HEADS, N, HID), den_ut,
                      b1_ut.reshape(1, HEADS * HID))
    h_us2 = _stage_b1(acc_tu.reshape(HEADS, N, HID), den_tu,
                      b1_tu.reshape(1, HEADS * HID))
    hs2_raw, as2_n, ad2_n = _stage_b2a(h_tx2, h_us2, W2_ut, as2_ut, ad2_ut)
    g2 = _gmax2(as2_n)
    hs2, meta2 = _stage_b2c(hs2_raw, as2_n, ad2_n, g2)

    acc2, den2 = _gat2(hs2, meta2, src_ut, dst_ut)
    den2 = den2.transpose(1, 0, 2).reshape(N, 2 * NC * NS)

    out = _stage_c(acc2, den2, b2_ut.reshape(1, OUT),
                   Wc.reshape(1, OUT), bc.reshape(1, 1))
    return out[:, 0]
